# baseline (speedup 1.0000x reference)
import functools

import jax
import jax.numpy as jnp
from jax import lax
from jax.experimental import pallas as pl
from jax.experimental.pallas import tpu as pltpu
from jax.experimental.pallas import tpu_sc as plsc

D = 64
BSTEP = 256
NW = 32
LANES = 16


@jax.jit
def _gather_t(xT, table):
    n_t, n_b = xT.shape
    b_per_w = n_b // NW
    halves = b_per_w // BSTEP
    n_steps = n_t * halves
    mesh = plsc.VectorSubcoreMesh(core_axis_name="c", subcore_axis_name="s")

    @functools.partial(
        pl.kernel,
        out_type=jax.ShapeDtypeStruct((n_t, D, n_b), jnp.float32),
        mesh=mesh,
        scratch_types=[
            pltpu.VMEM((BSTEP,), jnp.int32),
            pltpu.VMEM((BSTEP,), jnp.int32),
            pltpu.VMEM((BSTEP, D), jnp.float32),
            pltpu.VMEM((BSTEP, D), jnp.float32),
            pltpu.VMEM((D, BSTEP + 13), jnp.float32),
            pltpu.VMEM((D, BSTEP + 13), jnp.float32),
            pltpu.SemaphoreType.DMA,
            pltpu.SemaphoreType.DMA,
            pltpu.SemaphoreType.DMA,
            pltpu.SemaphoreType.DMA,
            pltpu.SemaphoreType.DMA,
            pltpu.SemaphoreType.DMA,
        ],
        compiler_params=pltpu.CompilerParams(use_tc_tiling_on_sc=False,
                                             needs_layout_passes=False),
    )
    def k(xT_hbm, table_hbm, p_hbm, idx_v0, idx_v1, rows_v0, rows_v1,
          tr_v0, tr_v1, sem_i0, sem_i1, sem_g0, sem_g1, sem_o0, sem_o1):
        idx_bufs = [idx_v0, idx_v1]
        row_bufs = [rows_v0, rows_v1]
        tr_bufs = [tr_v0, tr_v1]
        sem_i = [sem_i0, sem_i1]
        sem_g = [sem_g0, sem_g1]
        sem_o = [sem_o0, sem_o1]

        wid = lax.axis_index("s") * 2 + lax.axis_index("c")
        col0 = wid * b_per_w
        riota = lax.iota(jnp.int32, LANES)

        def idx_slice(s):
            return xT_hbm.at[s // halves,
                             pl.ds(col0 + (s % halves) * BSTEP, BSTEP)]

        def out_slice(s):
            return p_hbm.at[s // halves, :,
                            pl.ds(col0 + (s % halves) * BSTEP, BSTEP)]

        def tr_src(b):
            return tr_bufs[b].at[:, pl.ds(0, BSTEP)]

        def gather(b):
            return pltpu.make_async_copy(
                table_hbm.at[idx_bufs[b]], row_bufs[b], sem_g[b])

        def transpose(b):
            rows, tr = row_bufs[b], tr_bufs[b]

            def c_body(c4, carry):
                for u in range(4):
                    c = c4 * 4 + u
                    csplat = riota * 0 + c
                    for dg in range(D // LANES):
                        v = rows[c, pl.ds(dg * LANES, LANES)]
                        plsc.store_scatter(
                            tr, [riota + dg * LANES, csplat], v)
                return carry

            lax.fori_loop(0, BSTEP // 4, c_body, 0)

        for b in range(2):
            pltpu.async_copy(idx_slice(b), idx_bufs[b], sem_i[b])
        pltpu.make_async_copy(idx_slice(0), idx_bufs[0], sem_i[0]).wait()
        gather(0).start()

        def body(g2, carry):
            g = g2 * 2
            for b in range(2):
                s = g + b
                gather(b).wait()

                @pl.when(s + 1 < n_steps)
                def _next_gather():
                    pltpu.make_async_copy(
                        idx_slice(s + 1), idx_bufs[1 - b],
                        sem_i[1 - b]).wait()
                    gather(1 - b).start()

                @pl.when(s + 2 < n_steps)
                def _prefetch_idx():
                    pltpu.async_copy(idx_slice(s + 2), idx_bufs[b],
                                     sem_i[b])

                @pl.when(s >= 2)
                def _wait_out():
                    pltpu.make_async_copy(
                        tr_src(b), out_slice(s), sem_o[b]).wait()

                transpose(b)
                pltpu.async_copy(tr_src(b), out_slice(s), sem_o[b])
            return carry

        lax.fori_loop(0, n_steps // 2, body, 0)
        for b in range(2):
            pltpu.make_async_copy(
                tr_src(b), out_slice(n_steps - 2 + b), sem_o[b]).wait()

    return k(xT, table)


def kernel(x, table):
    p = _gather_t(x.T, table)
    return jnp.transpose(p, (2, 0, 1))


# speedup vs baseline: 1.4147x; 1.4147x over previous
import functools

import jax
import jax.numpy as jnp
from jax import lax
from jax.experimental import pallas as pl
from jax.experimental.pallas import tpu as pltpu
from jax.experimental.pallas import tpu_sc as plsc

D = 64
BSTEP = 256
NW = 32
LANES = 16


@jax.jit
def _gather_t(xT, table):
    n_t, n_b = xT.shape
    b_per_w = n_b // NW
    halves = b_per_w // BSTEP
    n_steps = n_t * halves
    mesh = plsc.VectorSubcoreMesh(core_axis_name="c", subcore_axis_name="s")

    @functools.partial(
        pl.kernel,
        out_type=jax.ShapeDtypeStruct((n_t, D, n_b), jnp.float32),
        mesh=mesh,
        scratch_types=[
            pltpu.VMEM((BSTEP,), jnp.int32),
            pltpu.VMEM((BSTEP,), jnp.int32),
            pltpu.VMEM((BSTEP, D), jnp.float32),
            pltpu.VMEM((BSTEP, D), jnp.float32),
            pltpu.VMEM((D, BSTEP + 1), jnp.float32),
            pltpu.VMEM((D, BSTEP + 1), jnp.float32),
            pltpu.SemaphoreType.DMA,
            pltpu.SemaphoreType.DMA,
            pltpu.SemaphoreType.DMA,
            pltpu.SemaphoreType.DMA,
            pltpu.SemaphoreType.DMA,
            pltpu.SemaphoreType.DMA,
        ],
        compiler_params=pltpu.CompilerParams(use_tc_tiling_on_sc=False,
                                             needs_layout_passes=False),
    )
    def k(xT_hbm, table_hbm, p_hbm, idx_v0, idx_v1, rows_v0, rows_v1,
          tr_v0, tr_v1, sem_i0, sem_i1, sem_g0, sem_g1, sem_o0, sem_o1):
        idx_bufs = [idx_v0, idx_v1]
        row_bufs = [rows_v0, rows_v1]
        tr_bufs = [tr_v0, tr_v1]
        sem_i = [sem_i0, sem_i1]
        sem_g = [sem_g0, sem_g1]
        sem_o = [sem_o0, sem_o1]

        wid = lax.axis_index("s") * 2 + lax.axis_index("c")
        col0 = wid * b_per_w
        riota = lax.iota(jnp.int32, LANES)

        def idx_slice(s):
            return xT_hbm.at[s // halves,
                             pl.ds(col0 + (s % halves) * BSTEP, BSTEP)]

        def out_slice(s):
            return p_hbm.at[s // halves, :,
                            pl.ds(col0 + (s % halves) * BSTEP, BSTEP)]

        def tr_src(b):
            return tr_bufs[b].at[:, pl.ds(0, BSTEP)]

        def gather(b):
            return pltpu.make_async_copy(
                table_hbm.at[idx_bufs[b]], row_bufs[b], sem_g[b])

        def transpose(b):
            rows, tr = row_bufs[b], tr_bufs[b]

            def c_body(c4, carry):
                vals = []
                for u in range(4):
                    c = c4 * 4 + u
                    for dg in range(D // LANES):
                        vals.append(rows[c, pl.ds(dg * LANES, LANES)])
                for u in range(4):
                    c = c4 * 4 + u
                    csplat = riota * 0 + c
                    for dg in range(D // LANES):
                        plsc.store_scatter(
                            tr, [riota + dg * LANES, csplat],
                            vals[u * (D // LANES) + dg])
                return carry

            lax.fori_loop(0, BSTEP // 4, c_body, 0)

        for b in range(2):
            pltpu.async_copy(idx_slice(b), idx_bufs[b], sem_i[b])
        pltpu.make_async_copy(idx_slice(0), idx_bufs[0], sem_i[0]).wait()
        gather(0).start()

        def body(g2, carry):
            g = g2 * 2
            for b in range(2):
                s = g + b
                gather(b).wait()

                @pl.when(s + 1 < n_steps)
                def _next_gather():
                    pltpu.make_async_copy(
                        idx_slice(s + 1), idx_bufs[1 - b],
                        sem_i[1 - b]).wait()
                    gather(1 - b).start()

                @pl.when(s + 2 < n_steps)
                def _prefetch_idx():
                    pltpu.async_copy(idx_slice(s + 2), idx_bufs[b],
                                     sem_i[b])

                @pl.when(s >= 2)
                def _wait_out():
                    pltpu.make_async_copy(
                        tr_src(b), out_slice(s), sem_o[b]).wait()

                transpose(b)
                pltpu.async_copy(tr_src(b), out_slice(s), sem_o[b])
            return carry

        lax.fori_loop(0, n_steps // 2, body, 0)
        for b in range(2):
            pltpu.make_async_copy(
                tr_src(b), out_slice(n_steps - 2 + b), sem_o[b]).wait()

    return k(xT, table)


def kernel(x, table):
    p = _gather_t(x.T, table)
    return jnp.transpose(p, (2, 0, 1))
